# baseline (speedup 1.0000x reference)
import jax
import jax.numpy as jnp
from jax.experimental import pallas as pl
from jax.experimental.pallas import tpu as pltpu

_N = 10000
_D_IN = 128
_D_OUT = 128
_BM = 200
_W2 = 2 * _N
_NBLK = _N // (2 * _BM)
_DEPTH = 2


def _a_copy(a_hbm, a_buf, sems, block, slot):
    return pltpu.make_async_copy(
        a_hbm.at[pl.ds(block * _BM, _BM), :], a_buf.at[slot], sems.at[slot]
    )


def _o_copy(o_stage, out_hbm, sems, block, slot):
    return pltpu.make_async_copy(
        o_stage.at[slot], out_hbm.at[pl.ds(block * _BM, _BM), :], sems.at[slot]
    )


def _gcn_kernel(x_ref, w_ref, b_ref, a_hbm, out_hbm,
                t2_ref, a_buf, o_stage, a_sems, o_sems):
    for k in range(_DEPTH):
        _a_copy(a_hbm, a_buf, a_sems, k, k).start()

    temple16 = jnp.dot(
        x_ref[...], w_ref[...],
        preferred_element_type=jnp.float32,
        precision=jax.lax.Precision.DEFAULT,
    ).astype(jnp.bfloat16)
    zeros16 = jnp.zeros((_N, _D_OUT), jnp.bfloat16)
    t2_ref[0:_N, 0:_D_OUT] = temple16
    t2_ref[0:_N, _D_OUT : 2 * _D_OUT] = zeros16
    t2_ref[_N : 2 * _N, 0:_D_OUT] = zeros16
    t2_ref[_N : 2 * _N, _D_OUT : 2 * _D_OUT] = temple16

    def body(i, carry):
        slot = jax.lax.rem(i, _DEPTH)
        oslot = jax.lax.rem(i, 2)

        @pl.when(i >= 2)
        def _drain_out():
            _o_copy(o_stage, out_hbm, o_sems, i - 2, oslot).wait()

        _a_copy(a_hbm, a_buf, a_sems, i, slot).wait()
        o_stage[oslot] = (
            jnp.dot(
                a_buf[slot], t2_ref[...],
                preferred_element_type=jnp.float32,
                precision=jax.lax.Precision.DEFAULT,
            )
            + b_ref[...]
        )

        @pl.when(i + _DEPTH < _NBLK)
        def _refill():
            _a_copy(a_hbm, a_buf, a_sems, i + _DEPTH, slot).start()

        _o_copy(o_stage, out_hbm, o_sems, i, oslot).start()
        return carry

    jax.lax.fori_loop(0, _NBLK, body, 0)

    for k in (_NBLK - 2, _NBLK - 1):
        _o_copy(o_stage, out_hbm, o_sems, k, k % 2).wait()


def kernel(adjacency, features_matrix, weights, bias):
    a2 = adjacency.reshape(_N // 2, _W2)
    bias2 = jnp.concatenate([bias, bias]).reshape(1, 2 * _D_OUT)
    out2 = pl.pallas_call(
        _gcn_kernel,
        in_specs=[
            pl.BlockSpec(memory_space=pltpu.MemorySpace.VMEM),
            pl.BlockSpec(memory_space=pltpu.MemorySpace.VMEM),
            pl.BlockSpec(memory_space=pltpu.MemorySpace.VMEM),
            pl.BlockSpec(memory_space=pltpu.MemorySpace.HBM),
        ],
        out_specs=pl.BlockSpec(memory_space=pltpu.MemorySpace.HBM),
        out_shape=jax.ShapeDtypeStruct((_N // 2, 2 * _D_OUT), jnp.float32),
        scratch_shapes=[
            pltpu.VMEM((_W2, 2 * _D_OUT), jnp.bfloat16),
            pltpu.VMEM((_DEPTH, _BM, _W2), jnp.float32),
            pltpu.VMEM((2, _BM, 2 * _D_OUT), jnp.float32),
            pltpu.SemaphoreType.DMA((_DEPTH,)),
            pltpu.SemaphoreType.DMA((2,)),
        ],
    )(features_matrix, weights, bias2, a2)
    return out2.reshape(_N, _D_OUT)


# speedup vs baseline: 4.0080x; 4.0080x over previous
import jax
import jax.numpy as jnp
from jax.experimental import pallas as pl
from jax.experimental.pallas import tpu as pltpu

_N = 10000
_D_IN = 128
_D_OUT = 128
_BM = 200
_NBLK = _N // _BM
_DEPTH = 3


def _a_copy(a_hbm, a_buf, sems, block, slot):
    return pltpu.make_async_copy(
        a_hbm.at[pl.ds(block * _BM, _BM), :], a_buf.at[slot], sems.at[slot]
    )


def _o_copy(o_stage, out_hbm, sems, block, slot):
    return pltpu.make_async_copy(
        o_stage.at[slot], out_hbm.at[pl.ds(block * _BM, _BM), :], sems.at[slot]
    )


def _gcn_kernel(x_ref, w_ref, b_ref, a_hbm, out_hbm,
                temple_ref, a_buf, o_stage, a_sems, o_sems):
    for k in range(_DEPTH):
        _a_copy(a_hbm, a_buf, a_sems, k, k).start()

    temple_ref[...] = jnp.dot(
        x_ref[...], w_ref[...],
        preferred_element_type=jnp.float32,
        precision=jax.lax.Precision.DEFAULT,
    )

    def body(i, carry):
        slot = jax.lax.rem(i, _DEPTH)
        oslot = jax.lax.rem(i, 2)

        @pl.when(i >= 2)
        def _drain_out():
            _o_copy(o_stage, out_hbm, o_sems, i - 2, oslot).wait()

        _a_copy(a_hbm, a_buf, a_sems, i, slot).wait()
        o_stage[oslot] = (
            jnp.dot(
                a_buf[slot], temple_ref[...],
                preferred_element_type=jnp.float32,
                precision=jax.lax.Precision.DEFAULT,
            )
            + b_ref[...]
        )

        @pl.when(i + _DEPTH < _NBLK)
        def _refill():
            _a_copy(a_hbm, a_buf, a_sems, i + _DEPTH, slot).start()

        _o_copy(o_stage, out_hbm, o_sems, i, oslot).start()
        return carry

    jax.lax.fori_loop(0, _NBLK, body, 0)

    for k in (_NBLK - 2, _NBLK - 1):
        _o_copy(o_stage, out_hbm, o_sems, k, k % 2).wait()


def kernel(adjacency, features_matrix, weights, bias):
    bias2d = bias.reshape(1, _D_OUT)
    return pl.pallas_call(
        _gcn_kernel,
        in_specs=[
            pl.BlockSpec(memory_space=pltpu.MemorySpace.VMEM),
            pl.BlockSpec(memory_space=pltpu.MemorySpace.VMEM),
            pl.BlockSpec(memory_space=pltpu.MemorySpace.VMEM),
            pl.BlockSpec(memory_space=pltpu.MemorySpace.HBM),
        ],
        out_specs=pl.BlockSpec(memory_space=pltpu.MemorySpace.HBM),
        out_shape=jax.ShapeDtypeStruct((_N, _D_OUT), jnp.float32),
        scratch_shapes=[
            pltpu.VMEM((_N, _D_IN), jnp.float32),
            pltpu.VMEM((_DEPTH, _BM, _N), jnp.float32),
            pltpu.VMEM((2, _BM, _D_OUT), jnp.float32),
            pltpu.SemaphoreType.DMA((_DEPTH,)),
            pltpu.SemaphoreType.DMA((2,)),
        ],
    )(features_matrix, weights, bias2d, adjacency)
